# baseline (speedup 1.0000x reference)
import jax
import jax.numpy as jnp
import numpy as np
from jax import lax
from jax.experimental import pallas as pl
from jax.experimental.pallas import tpu as pltpu
from jax.experimental.pallas import tpu_sc as plsc


def kernel(frames):
    C, T, H, W = frames.shape
    S = T // 4

    idx = np.linspace(0.0, T - 1, S).astype(np.int64)
    jt = (np.arange(T) * (S - 1) + (T - 2)) // (T - 1)
    assert np.array_equal(jt, np.searchsorted(idx, np.arange(T)))
    assert np.all(np.diff(idx) > 0)

    ROW = H * W
    N = C * T
    NC, NS = 2, 16
    NW = NC * NS
    assert N % NW == 0
    RPW = N // NW
    CHUNK = ROW // 2
    assert ROW % CHUNK == 0 and CHUNK % 8 == 0
    CPR = ROW // CHUNK
    NCH = RPW * CPR

    x = frames.reshape(-1)

    mesh = plsc.VectorSubcoreMesh(
        core_axis_name="c", subcore_axis_name="s")

    def body(x_hbm, fast_hbm, slow_hbm, buf, insem, outsem):
        wid = lax.axis_index("s") * NC + lax.axis_index("c")

        def info(k):
            row = wid * RPW + (k // CPR)
            t = lax.rem(row, T)
            ch = lax.div(row, T)
            j0 = (t * (S - 1) + (T - 2)) // (T - 1)
            j1 = ((t + 1) * (S - 1) + (T - 2)) // (T - 1)
            issel = jnp.logical_or(t == T - 1, j1 != j0)
            off = row * ROW + (k % CPR) * CHUNK
            soff = (ch * S + j0) * ROW + (k % CPR) * CHUNK
            return off, issel, soff

        def in_cp(k, s):
            off, _, _ = info(k)
            return pltpu.make_async_copy(
                x_hbm.at[pl.ds(pl.multiple_of(off, 8), CHUNK)],
                buf.at[s], insem.at[s])

        def fast_cp(k, s):
            off, _, _ = info(k)
            return pltpu.make_async_copy(
                buf.at[s], fast_hbm.at[pl.ds(pl.multiple_of(off, 8), CHUNK)],
                outsem.at[s])

        def slow_cp(k, s):
            _, _, soff = info(k)
            return pltpu.make_async_copy(
                buf.at[s], slow_hbm.at[pl.ds(pl.multiple_of(soff, 8), CHUNK)],
                outsem.at[s])

        def out_wait(k, s):
            fast_cp(k, s).wait()
            _, issel, _ = info(k)

            @pl.when(issel)
            def _():
                slow_cp(k, s).wait()

        in_cp(0, 0).start()
        for k in range(NCH):
            s = k % 2
            in_cp(k, s).wait()
            fast_cp(k, s).start()
            _, issel, _ = info(k)

            @pl.when(issel)
            def _():
                slow_cp(k, s).start()

            if k + 1 < NCH:
                if k >= 1:
                    out_wait(k - 1, (k - 1) % 2)
                in_cp(k + 1, (k + 1) % 2).start()
        for k in range(max(NCH - 2, 0), NCH):
            out_wait(k, k % 2)

    run = pl.kernel(
        body,
        out_type=[
            jax.ShapeDtypeStruct((N * ROW,), frames.dtype),
            jax.ShapeDtypeStruct((C * S * ROW,), frames.dtype),
        ],
        mesh=mesh,
        scratch_types=[
            pltpu.VMEM((2, CHUNK), frames.dtype),
            pltpu.SemaphoreType.DMA((2,)),
            pltpu.SemaphoreType.DMA((2,)),
        ],
    )
    fast, slow = run(x)
    return (slow.reshape(C, S, H, W), fast.reshape(C, T, H, W))
